# baseline (speedup 1.0000x reference)
import jax
import jax.numpy as jnp
from jax.experimental import pallas as pl
from jax.experimental.pallas import tpu as pltpu

B, H, W, C = 64, 32, 32, 768
N = B * H * W
LANES = 128
SUB = C // LANES
ROWS = 2048
GRID = N // ROWS


def _cam_body(x_ref, w_ref, o_ref):
    x = x_ref[...].reshape(ROWS, SUB, LANES)
    w = w_ref[...].reshape(1, SUB, LANES)
    o_ref[...] = jnp.sum(x * w, axis=(1, 2), keepdims=False).reshape(ROWS, 1)


def kernel(conv_input, output, weight):
    x = conv_input.reshape(N * SUB, LANES)
    w = weight.reshape(SUB, LANES)
    out = pl.pallas_call(
        _cam_body,
        grid=(GRID,),
        in_specs=[
            pl.BlockSpec((ROWS * SUB, LANES), lambda i: (i, 0)),
            pl.BlockSpec((SUB, LANES), lambda i: (0, 0)),
        ],
        out_specs=pl.BlockSpec((ROWS, 1), lambda i: (i, 0)),
        out_shape=jax.ShapeDtypeStruct((N, 1), jnp.float32),
    )(x, w)
    return (out.reshape(B, H, W), output)


# speedup vs baseline: 5.6918x; 5.6918x over previous
import jax
import jax.numpy as jnp
from jax.experimental import pallas as pl
from jax.experimental.pallas import tpu as pltpu

B, H, W, C = 64, 32, 32, 768
N = B * H * W
LANES = 128
ROWS = 2048
GRID = N // ROWS


def _cam_body(x_ref, w_ref, o_ref):
    r = jnp.sum(x_ref[...] * w_ref[...], axis=1)
    o_ref[...] = r.reshape(ROWS // LANES, LANES)


def kernel(conv_input, output, weight):
    x = conv_input.reshape(N, C)
    w = weight.reshape(1, C)
    out = pl.pallas_call(
        _cam_body,
        grid=(GRID,),
        in_specs=[
            pl.BlockSpec((ROWS, C), lambda i: (i, 0)),
            pl.BlockSpec((1, C), lambda i: (0, 0)),
        ],
        out_specs=pl.BlockSpec((ROWS // LANES, LANES), lambda i: (i, 0)),
        out_shape=jax.ShapeDtypeStruct((N // LANES, LANES), jnp.float32),
    )(x, w)
    return (out.reshape(B, H, W), output)
